# baseline (speedup 1.0000x reference)
import jax
import jax.numpy as jnp
from jax.experimental import pallas as pl
from jax.experimental.pallas import tpu as pltpu


def _gcn_matmul_kernel(adj_ref, emb_ref, out_ref):
    a = adj_ref[...].astype(jnp.bfloat16)
    e = emb_ref[...]
    e_hi = e.astype(jnp.bfloat16)
    e_lo = (e - e_hi.astype(jnp.float32)).astype(jnp.bfloat16)
    out_ref[...] = jnp.dot(
        a, e_hi, preferred_element_type=jnp.float32
    ) + jnp.dot(a, e_lo, preferred_element_type=jnp.float32)


def kernel(adj, embeds, batch_size):
    adj = adj.astype(jnp.float32)
    embeds = embeds.astype(jnp.float32)
    n, k = adj.shape
    d = embeds.shape[1]
    bm = 512
    return pl.pallas_call(
        _gcn_matmul_kernel,
        grid=(n // bm,),
        compiler_params=pltpu.CompilerParams(
            dimension_semantics=("parallel",)
        ),
        in_specs=[
            pl.BlockSpec((bm, k), lambda i: (i, 0)),
            pl.BlockSpec((k, d), lambda i: (0, 0)),
        ],
        out_specs=pl.BlockSpec((bm, d), lambda i: (i, 0)),
        out_shape=jax.ShapeDtypeStruct((n, d), jnp.float32),
    )(adj, embeds)


# speedup vs baseline: 1.1344x; 1.1344x over previous
import jax
import jax.numpy as jnp
from jax.experimental import pallas as pl
from jax.experimental.pallas import tpu as pltpu


def _gcn_matmul_kernel(adj_ref, emb_ref, out_ref):
    out_ref[...] = jnp.dot(
        adj_ref[...], emb_ref[...], preferred_element_type=jnp.float32
    )


def kernel(adj, embeds, batch_size):
    adj = adj.astype(jnp.float32)
    embeds = embeds.astype(jnp.float32)
    n, k = adj.shape
    d = embeds.shape[1]
    bm = 512
    return pl.pallas_call(
        _gcn_matmul_kernel,
        grid=(n // bm,),
        compiler_params=pltpu.CompilerParams(
            dimension_semantics=("parallel",)
        ),
        in_specs=[
            pl.BlockSpec((bm, k), lambda i: (i, 0)),
            pl.BlockSpec((k, d), lambda i: (0, 0)),
        ],
        out_specs=pl.BlockSpec((bm, d), lambda i: (i, 0)),
        out_shape=jax.ShapeDtypeStruct((n, d), jnp.float32),
    )(adj, embeds)
